# baseline (speedup 1.0000x reference)
import jax
import jax.numpy as jnp
from jax import lax
from jax.experimental import pallas as pl
from jax.experimental.pallas import tpu as pltpu
from jax.experimental.pallas import tpu_sc as plsc

R = 32
A = 8
N = 4096
EPS = 1e-12
NC = 2
NS = 16
L = 16
NW = NC * NS
SPW = N // NW
NB = SPW // L
PF = 264
PAD = 8


def _bcast(ref, i):
    return plsc.load_gather(ref, [jnp.full((L,), i, jnp.int32)])


def _sc_body(xt_hbm, frb_hbm, c1_hbm, c2_hbm, out_hbm,
             x_v, f_v, c1_v, c2_v, m_v, wu_v, wl_v,
             b2_v, b1_v, rk1_v, rk2_v, d1_v, d2_v, out_v):
    wid = lax.axis_index("s") * NC + lax.axis_index("c")
    base = wid * SPW
    pltpu.sync_copy(xt_hbm.at[:, pl.ds(base, SPW)], x_v)
    pltpu.sync_copy(frb_hbm.at[pl.ds(0, PF)], f_v)
    pltpu.sync_copy(c1_hbm, c1_v.at[pl.ds(PAD, R)])
    pltpu.sync_copy(c2_hbm, c2_v.at[pl.ds(PAD, R)])
    iota = lax.iota(jnp.int32, L)

    for chunk in range(R * A // L):
        b0 = chunk * L
        mv = f_v[pl.ds(b0, L)]
        sav = plsc.load_gather(f_v, [iota + (b0 + 1)])
        sbv = plsc.load_gather(f_v, [iota + (b0 + 2)])
        sbig = jnp.maximum(sav, sbv)
        ssm = jnp.minimum(sav, sbv)
        m_v[pl.ds(b0 + PAD, L)] = mv
        wu_v[pl.ds(b0 + PAD, L)] = -0.5 / (sbig * sbig)
        wl_v[pl.ds(b0 + PAD, L)] = -0.5 / (ssm * ssm)

    for c_v, b_v, rk_v in ((c1_v, b2_v, rk1_v), (c2_v, b1_v, rk2_v)):
        ci0 = c_v[pl.ds(PAD, L)]
        ci1 = c_v[pl.ds(PAD + L, L)]
        cnt0 = jnp.zeros((L,), jnp.int32)
        cnt1 = jnp.zeros((L,), jnp.int32)
        for j in range(R):
            cj = _bcast(c_v, j + PAD)
            win0 = (cj < ci0) | ((cj == ci0) & (j < iota))
            win1 = (cj < ci1) | ((cj == ci1) & (j < iota + L))
            cnt0 = cnt0 + jnp.where(win0, 1, 0)
            cnt1 = cnt1 + jnp.where(win1, 1, 0)
        rk_v[pl.ds(0, L)] = cnt0
        rk_v[pl.ds(L, L)] = cnt1
        plsc.store_scatter(b_v, [cnt0], ci0)
        plsc.store_scatter(b_v, [cnt1], ci1)

    def blk_body(blk, carry):
        col = blk * L + iota
        xs = [plsc.load_gather(x_v, [jnp.full((L,), a, jnp.int32), col])
              for a in range(A)]
        c1c = (c1_v[pl.ds(PAD, L)], c1_v[pl.ds(PAD + L, L)])
        c2c = (c2_v[pl.ds(PAD, L)], c2_v[pl.ds(PAD + L, L)])
        rk1c = (rk1_v[pl.ds(0, L)], rk1_v[pl.ds(L, L)])
        rk2c = (rk2_v[pl.ds(0, L)], rk2_v[pl.ds(L, L)])

        s0l = jnp.zeros((L,), jnp.float32)
        t0l = jnp.zeros((L,), jnp.float32)
        s0r = jnp.zeros((L,), jnp.float32)
        t0r = jnp.zeros((L,), jnp.float32)
        for r in range(R):
            au = None
            al = None
            for a in range(A):
                p = r * A + a + PAD
                d = xs[a] - _bcast(m_v, p)
                d2 = d * d
                if a == 0:
                    au = d2 * _bcast(wu_v, p)
                    al = d2 * _bcast(wl_v, p)
                else:
                    au = au + d2 * _bcast(wu_v, p)
                    al = al + d2 * _bcast(wl_v, p)
            uu = jnp.exp(au)
            ll = jnp.exp(al)
            hi, lo = r // L, r % L
            s0l = s0l + c1c[hi][lo] * ll
            t0l = t0l + ll
            s0r = s0r + c2c[hi][lo] * uu
            t0r = t0r + uu
            dlt = uu - ll
            plsc.store_scatter(d1_v, [rk1c[hi][lo] * L + iota], dlt)
            plsc.store_scatter(d2_v, [rk2c[hi][lo] * L + iota], dlt)

        b2c = (b2_v[pl.ds(0, L)], b2_v[pl.ds(L, L)])
        lmin = s0l / (t0l + EPS)
        s = s0l
        t = t0l
        for k in range(R):
            dk = d1_v[pl.ds(k * L, L)]
            s = s + b2c[k // L][k % L] * dk
            t = t + dk
            lmin = jnp.minimum(lmin, s / (t + EPS))

        b1c = (b1_v[pl.ds(0, L)], b1_v[pl.ds(L, L)])
        rmax = s0r / (t0r + EPS)
        s = s0r
        t = t0r
        for k in range(R):
            dk = d2_v[pl.ds(k * L, L)]
            s = s - b1c[k // L][k % L] * dk
            t = t - dk
            rmax = jnp.maximum(rmax, s / (t + EPS))

        plsc.store_scatter(out_v, [col], (lmin + rmax) * 0.5)
        return carry

    lax.fori_loop(0, NB, blk_body, 0)
    pltpu.sync_copy(out_v, out_hbm.at[pl.ds(base, SPW)])


_km_kernel = pl.kernel(
    _sc_body,
    out_type=jax.ShapeDtypeStruct((N,), jnp.float32),
    mesh=plsc.VectorSubcoreMesh(
        core_axis_name="c", subcore_axis_name="s",
        num_cores=NC, num_subcores=NS),
    compiler_params=pltpu.CompilerParams(needs_layout_passes=False),
    scratch_types=[
        pltpu.VMEM((A, SPW), jnp.float32),
        pltpu.VMEM((PF,), jnp.float32),
        pltpu.VMEM((R + PAD,), jnp.float32),
        pltpu.VMEM((R + PAD,), jnp.float32),
        pltpu.VMEM((R * A + PAD,), jnp.float32),
        pltpu.VMEM((R * A + PAD,), jnp.float32),
        pltpu.VMEM((R * A + PAD,), jnp.float32),
        pltpu.VMEM((R,), jnp.float32),
        pltpu.VMEM((R,), jnp.float32),
        pltpu.VMEM((R,), jnp.int32),
        pltpu.VMEM((R,), jnp.int32),
        pltpu.VMEM((R * L,), jnp.float32),
        pltpu.VMEM((R * L,), jnp.float32),
        pltpu.VMEM((SPW,), jnp.float32),
    ],
)


@jax.jit
def kernel(input_data, FRB_weights, c1, c2):
    return _km_kernel(input_data.T, FRB_weights, c1, c2)


# speedup vs baseline: 1.2634x; 1.2634x over previous
import jax
import jax.numpy as jnp
from jax import lax
from jax.experimental import pallas as pl
from jax.experimental.pallas import tpu as pltpu
from jax.experimental.pallas import tpu_sc as plsc

R = 32
A = 8
N = 4096
EPS = 1e-12
NC = 2
NS = 16
L = 16
NW = NC * NS
SPW = N // NW
NB = SPW // L
PF = 264
PAD = 8


def _bcast(ref, i):
    return plsc.load_gather(ref, [jnp.full((L,), i, jnp.int32)])


def _sc_body(xt_hbm, frb_hbm, c1_hbm, c2_hbm, out_hbm,
             x_v, f_v, c1_v, c2_v, m_v, wu_v, wl_v,
             b2_v, b1_v, rk1_v, rk2_v, d1_v, d2_v, out_v):
    wid = lax.axis_index("s") * NC + lax.axis_index("c")
    base = wid * SPW
    pltpu.sync_copy(xt_hbm.at[:, pl.ds(base, SPW)], x_v)
    pltpu.sync_copy(frb_hbm.at[pl.ds(0, PF)], f_v)
    pltpu.sync_copy(c1_hbm, c1_v.at[pl.ds(PAD, R)])
    pltpu.sync_copy(c2_hbm, c2_v.at[pl.ds(PAD, R)])
    iota = lax.iota(jnp.int32, L)

    for chunk in range(R * A // L):
        b0 = chunk * L
        mv = f_v[pl.ds(b0, L)]
        sav = plsc.load_gather(f_v, [iota + (b0 + 1)])
        sbv = plsc.load_gather(f_v, [iota + (b0 + 2)])
        sbig = jnp.maximum(sav, sbv)
        ssm = jnp.minimum(sav, sbv)
        m_v[pl.ds(b0 + PAD, L)] = mv
        wu_v[pl.ds(b0 + PAD, L)] = -0.5 / (sbig * sbig)
        wl_v[pl.ds(b0 + PAD, L)] = -0.5 / (ssm * ssm)

    for c_v, b_v, rk_v in ((c1_v, b2_v, rk1_v), (c2_v, b1_v, rk2_v)):
        ci0 = c_v[pl.ds(PAD, L)]
        ci1 = c_v[pl.ds(PAD + L, L)]
        cnt0 = jnp.zeros((L,), jnp.int32)
        cnt1 = jnp.zeros((L,), jnp.int32)
        for j in range(R):
            cj = (ci0, ci1)[j // L][j % L]
            win0 = (cj < ci0) | ((cj == ci0) & (j < iota))
            win1 = (cj < ci1) | ((cj == ci1) & (j < iota + L))
            cnt0 = cnt0 + jnp.where(win0, 1, 0)
            cnt1 = cnt1 + jnp.where(win1, 1, 0)
        rk_v[pl.ds(0, L)] = cnt0
        rk_v[pl.ds(L, L)] = cnt1
        plsc.store_scatter(b_v, [cnt0], ci0)
        plsc.store_scatter(b_v, [cnt1], ci1)

    def blk_body(blk, carry):
        col = blk * L + iota
        xs = [plsc.load_gather(x_v, [jnp.full((L,), a, jnp.int32), col])
              for a in range(A)]
        c1c = (c1_v[pl.ds(PAD, L)], c1_v[pl.ds(PAD + L, L)])
        c2c = (c2_v[pl.ds(PAD, L)], c2_v[pl.ds(PAD + L, L)])
        rk1c = (rk1_v[pl.ds(0, L)], rk1_v[pl.ds(L, L)])
        rk2c = (rk2_v[pl.ds(0, L)], rk2_v[pl.ds(L, L)])

        s0l = jnp.zeros((L,), jnp.float32)
        t0l = jnp.zeros((L,), jnp.float32)
        s0r = jnp.zeros((L,), jnp.float32)
        t0r = jnp.zeros((L,), jnp.float32)
        for r in range(R):
            if r % 2 == 0:
                cb = PAD + (r // 2) * L
                mch = m_v[pl.ds(cb, L)]
                wuch = wu_v[pl.ds(cb, L)]
                wlch = wl_v[pl.ds(cb, L)]
            au = None
            al = None
            for a in range(A):
                q = (r % 2) * A + a
                d = xs[a] - mch[q]
                d2 = d * d
                if a == 0:
                    au = d2 * wuch[q]
                    al = d2 * wlch[q]
                else:
                    au = au + d2 * wuch[q]
                    al = al + d2 * wlch[q]
            uu = jnp.exp(au)
            ll = jnp.exp(al)
            hi, lo = r // L, r % L
            s0l = s0l + c1c[hi][lo] * ll
            t0l = t0l + ll
            s0r = s0r + c2c[hi][lo] * uu
            t0r = t0r + uu
            dlt = uu - ll
            plsc.store_scatter(d1_v, [rk1c[hi][lo] * L + iota], dlt)
            plsc.store_scatter(d2_v, [rk2c[hi][lo] * L + iota], dlt)

        b2c = (b2_v[pl.ds(0, L)], b2_v[pl.ds(L, L)])
        lmin = s0l / (t0l + EPS)
        s = s0l
        t = t0l
        for k in range(R):
            dk = d1_v[pl.ds(k * L, L)]
            s = s + b2c[k // L][k % L] * dk
            t = t + dk
            lmin = jnp.minimum(lmin, s / (t + EPS))

        b1c = (b1_v[pl.ds(0, L)], b1_v[pl.ds(L, L)])
        rmax = s0r / (t0r + EPS)
        s = s0r
        t = t0r
        for k in range(R):
            dk = d2_v[pl.ds(k * L, L)]
            s = s - b1c[k // L][k % L] * dk
            t = t - dk
            rmax = jnp.maximum(rmax, s / (t + EPS))

        plsc.store_scatter(out_v, [col], (lmin + rmax) * 0.5)
        return carry

    lax.fori_loop(0, NB, blk_body, 0)
    pltpu.sync_copy(out_v, out_hbm.at[pl.ds(base, SPW)])


_km_kernel = pl.kernel(
    _sc_body,
    out_type=jax.ShapeDtypeStruct((N,), jnp.float32),
    mesh=plsc.VectorSubcoreMesh(
        core_axis_name="c", subcore_axis_name="s",
        num_cores=NC, num_subcores=NS),
    compiler_params=pltpu.CompilerParams(needs_layout_passes=False),
    scratch_types=[
        pltpu.VMEM((A, SPW), jnp.float32),
        pltpu.VMEM((PF,), jnp.float32),
        pltpu.VMEM((R + PAD,), jnp.float32),
        pltpu.VMEM((R + PAD,), jnp.float32),
        pltpu.VMEM((R * A + PAD,), jnp.float32),
        pltpu.VMEM((R * A + PAD,), jnp.float32),
        pltpu.VMEM((R * A + PAD,), jnp.float32),
        pltpu.VMEM((R,), jnp.float32),
        pltpu.VMEM((R,), jnp.float32),
        pltpu.VMEM((R,), jnp.int32),
        pltpu.VMEM((R,), jnp.int32),
        pltpu.VMEM((R * L,), jnp.float32),
        pltpu.VMEM((R * L,), jnp.float32),
        pltpu.VMEM((SPW,), jnp.float32),
    ],
)


@jax.jit
def kernel(input_data, FRB_weights, c1, c2):
    return _km_kernel(input_data.T, FRB_weights, c1, c2)
